# baseline (speedup 1.0000x reference)
import functools

import jax
import jax.numpy as jnp
from jax import lax
from jax.experimental import pallas as pl
from jax.experimental.pallas import tpu as pltpu
from jax.experimental.pallas import tpu_sc as plsc

N, D = 4096, 2048
NC, NS, L = 2, 16, 16
NW = NC * NS
RPW = N // NW
NB = 8
NBLK = RPW // NB

_mesh = plsc.VectorSubcoreMesh(core_axis_name="c", subcore_axis_name="s")


@functools.partial(
    pl.kernel,
    out_type=jax.ShapeDtypeStruct((N, D), jnp.float32),
    mesh=_mesh,
    scratch_types=[
        pltpu.VMEM((RPW,), jnp.int32),
        pltpu.VMEM((L,), jnp.float32),
        pltpu.VMEM((NB, D), jnp.float32),
        pltpu.VMEM((NB, D), jnp.float32),
        pltpu.SemaphoreType.DMA,
        pltpu.SemaphoreType.DMA,
    ],
)
def _mix_sc(x_hbm, idx_hbm, rat_hbm, out_hbm, idx_v, rat_v, lin_v, mix_v,
            sem_l, sem_m):
    wid = lax.axis_index("s") * NC + lax.axis_index("c")
    base = wid * RPW
    pltpu.sync_copy(idx_hbm.at[pl.ds(base, RPW)], idx_v)
    pltpu.sync_copy(rat_hbm, rat_v)
    r = rat_v[...]
    om = 1.0 - r
    for g in range(NBLK):
        cl = pltpu.async_copy(x_hbm.at[pl.ds(base + g * NB, NB)], lin_v, sem_l)
        cm = pltpu.async_copy(x_hbm.at[idx_v.at[pl.ds(g * NB, NB)]], mix_v,
                              sem_m)
        cl.wait()
        cm.wait()
        for i in range(NB):
            def blend(j, _, i=i):
                a = lin_v[i, pl.ds(j * L, L)]
                b = mix_v[i, pl.ds(j * L, L)]
                lin_v[i, pl.ds(j * L, L)] = om * a + r * b
                return _
            lax.fori_loop(0, D // L, blend, None)
        pltpu.sync_copy(lin_v, out_hbm.at[pl.ds(base + g * NB, NB)])


def kernel(x, index, ratio):
    idx32 = index.astype(jnp.int32)
    rat16 = jnp.broadcast_to(ratio.astype(jnp.float32), (L,))
    return _mix_sc(x, idx32, rat16)


# speedup vs baseline: 1.2064x; 1.2064x over previous
import functools

import jax
import jax.numpy as jnp
from jax import lax
from jax.experimental import pallas as pl
from jax.experimental.pallas import tpu as pltpu
from jax.experimental.pallas import tpu_sc as plsc

N, D = 4096, 2048
NC, NS, L = 2, 16, 16
NW = NC * NS
RPW = N // NW
NB = 8
NBLK = RPW // NB
NSLOT = 2

_mesh = plsc.VectorSubcoreMesh(core_axis_name="c", subcore_axis_name="s")


@functools.partial(
    pl.kernel,
    out_type=jax.ShapeDtypeStruct((N, D), jnp.float32),
    mesh=_mesh,
    scratch_types=[
        pltpu.VMEM((RPW,), jnp.int32),
        pltpu.VMEM((L,), jnp.float32),
        pltpu.VMEM((NSLOT, NB, D), jnp.float32),
        pltpu.VMEM((NSLOT, NB, D), jnp.float32),
        pltpu.SemaphoreType.DMA,
        pltpu.SemaphoreType.DMA,
        pltpu.SemaphoreType.DMA,
        pltpu.SemaphoreType.DMA,
        pltpu.SemaphoreType.DMA,
        pltpu.SemaphoreType.DMA,
    ],
)
def _mix_sc(x_hbm, idx_hbm, rat_hbm, out_hbm, idx_v, rat_v, lin_v, mix_v,
            sl0, sl1, sm0, sm1, ss0, ss1):
    sem_lin = (sl0, sl1)
    sem_mix = (sm0, sm1)
    sem_out = (ss0, ss1)
    wid = lax.axis_index("s") * NC + lax.axis_index("c")
    base = wid * RPW
    pltpu.sync_copy(idx_hbm.at[pl.ds(base, RPW)], idx_v)
    pltpu.sync_copy(rat_hbm, rat_v)
    r = rat_v[...]
    om = 1.0 - r

    def start_loads(g):
        s = g % NSLOT
        dl = pltpu.async_copy(x_hbm.at[pl.ds(base + g * NB, NB)],
                              lin_v.at[s], sem_lin[s])
        dm = pltpu.async_copy(x_hbm.at[idx_v.at[pl.ds(g * NB, NB)]],
                              mix_v.at[s], sem_mix[s])
        return dl, dm

    loads = [None, None]
    stores = [None, None]
    loads[0] = start_loads(0)
    for g in range(NBLK):
        s = g % NSLOT
        ns = (g + 1) % NSLOT
        if g + 1 < NBLK:
            if stores[ns] is not None:
                stores[ns].wait()
                stores[ns] = None
            loads[ns] = start_loads(g + 1)
        dl, dm = loads[s]
        dl.wait()
        dm.wait()
        for i in range(NB):
            def blend(j, _, s=s, i=i):
                a = lin_v[s, i, pl.ds(j * L, L)]
                b = mix_v[s, i, pl.ds(j * L, L)]
                lin_v[s, i, pl.ds(j * L, L)] = om * a + r * b
                return _
            lax.fori_loop(0, D // L, blend, None)
        stores[s] = pltpu.async_copy(lin_v.at[s],
                                     out_hbm.at[pl.ds(base + g * NB, NB)],
                                     sem_out[s])
    for s in range(NSLOT):
        if stores[s] is not None:
            stores[s].wait()


def kernel(x, index, ratio):
    idx32 = index.astype(jnp.int32)
    rat16 = jnp.broadcast_to(ratio.astype(jnp.float32), (L,))
    return _mix_sc(x, idx32, rat16)


# speedup vs baseline: 2.3741x; 1.9679x over previous
import functools

import jax
import jax.numpy as jnp
from jax import lax
from jax.experimental import pallas as pl
from jax.experimental.pallas import tpu as pltpu
from jax.experimental.pallas import tpu_sc as plsc

N, D = 4096, 2048
NC, NS, L = 2, 16, 16
NW = NC * NS
RPW = N // NW
NB = 8
NBLK = RPW // NB
NSLOT = 2

_mesh = plsc.VectorSubcoreMesh(core_axis_name="c", subcore_axis_name="s")


@functools.partial(
    pl.kernel,
    out_type=jax.ShapeDtypeStruct((N, D), jnp.float32),
    mesh=_mesh,
    scratch_types=[
        pltpu.VMEM((RPW,), jnp.int32),
        pltpu.VMEM((L,), jnp.float32),
        pltpu.VMEM((NSLOT, NB, D), jnp.float32),
        pltpu.VMEM((NSLOT, NB, D), jnp.float32),
        pltpu.SemaphoreType.DMA,
        pltpu.SemaphoreType.DMA,
        pltpu.SemaphoreType.DMA,
        pltpu.SemaphoreType.DMA,
        pltpu.SemaphoreType.DMA,
        pltpu.SemaphoreType.DMA,
    ],
)
def _mix_sc(x_hbm, idx_hbm, rat_hbm, out_hbm, idx_v, rat_v, lin_v, mix_v,
            sl0, sl1, sm0, sm1, ss0, ss1):
    sem_lin = (sl0, sl1)
    sem_mix = (sm0, sm1)
    sem_out = (ss0, ss1)
    wid = lax.axis_index("s") * NC + lax.axis_index("c")
    base = wid * RPW
    pltpu.sync_copy(idx_hbm.at[pl.ds(base, RPW)], idx_v)
    pltpu.sync_copy(rat_hbm, rat_v)
    r = rat_v[...]
    om = 1.0 - r

    def start_loads(g):
        s = g % NSLOT
        dl = pltpu.async_copy(x_hbm.at[pl.ds(base + g * NB, NB)],
                              lin_v.at[s], sem_lin[s])
        dm = pltpu.async_copy(x_hbm.at[idx_v.at[pl.ds(g * NB, NB)]],
                              mix_v.at[s], sem_mix[s])
        return dl, dm

    loads = [None, None]
    stores = [None, None]
    loads[0] = start_loads(0)
    for g in range(NBLK):
        s = g % NSLOT
        ns = (g + 1) % NSLOT
        if g + 1 < NBLK:
            if stores[ns] is not None:
                stores[ns].wait()
                stores[ns] = None
            loads[ns] = start_loads(g + 1)
        dl, dm = loads[s]
        dl.wait()
        dm.wait()
        for i in range(NB):
            @plsc.parallel_loop(0, D, step=L, unroll=8)
            def blend(j, s=s, i=i):
                a = lin_v[s, i, pl.ds(j, L)]
                b = mix_v[s, i, pl.ds(j, L)]
                lin_v[s, i, pl.ds(j, L)] = om * a + r * b
        stores[s] = pltpu.async_copy(lin_v.at[s],
                                     out_hbm.at[pl.ds(base + g * NB, NB)],
                                     sem_out[s])
    for s in range(NSLOT):
        if stores[s] is not None:
            stores[s].wait()


def kernel(x, index, ratio):
    idx32 = index.astype(jnp.int32)
    rat16 = jnp.broadcast_to(ratio.astype(jnp.float32), (L,))
    return _mix_sc(x, idx32, rat16)
